# baseline (speedup 1.0000x reference)
import functools
import math

import jax
import jax.numpy as jnp
from jax import lax
from jax.experimental import pallas as pl
from jax.experimental.pallas import tpu as pltpu
from jax.experimental.pallas import tpu_sc as plsc

D_MODEL = 1024
SCALE = math.sqrt(D_MODEL)
L = 16
NC, NS = 2, 16
NW = NC * NS
B_TOTAL = 4 * 4096
BPW = B_TOTAL // NW
C = 32
NCHUNK = BPW // C
NBUF = 3

_mesh = plsc.VectorSubcoreMesh(core_axis_name="c", subcore_axis_name="s")


@functools.partial(
    pl.kernel,
    mesh=_mesh,
    out_type=jax.ShapeDtypeStruct((B_TOTAL, D_MODEL), jnp.float32),
    scratch_types=[
        pltpu.VMEM((BPW,), jnp.int32),
        pltpu.VMEM((NBUF, C, D_MODEL), jnp.float32),
        pltpu.SemaphoreType.DMA,
        pltpu.SemaphoreType.DMA,
    ],
)
def _emb_lookup(table_hbm, idx_hbm, out_hbm, idx_v, rows_v, gsem, osem):
    wid = lax.axis_index("s") * NC + lax.axis_index("c")
    base = wid * BPW

    pltpu.sync_copy(idx_hbm.at[pl.ds(base, BPW)], idx_v)

    def gather_start(c, b):
        return pltpu.async_copy(
            table_hbm.at[idx_v.at[pl.ds(c * C, C)]], rows_v.at[b], gsem
        )

    def gather_wait(c, b):
        pltpu.make_async_copy(
            table_hbm.at[idx_v.at[pl.ds(c * C, C)]], rows_v.at[b], gsem
        ).wait()

    def out_start(c, b):
        return pltpu.async_copy(
            rows_v.at[b], out_hbm.at[pl.ds(base + c * C, C)], osem
        )

    def out_wait(c, b):
        pltpu.make_async_copy(
            rows_v.at[b], out_hbm.at[pl.ds(base + c * C, C)], osem
        ).wait()

    def scale_buf(b):
        def row_body(r, _):
            for j in range(D_MODEL // L):
                sl = pl.ds(j * L, L)
                rows_v[b, r, sl] = rows_v[b, r, sl] * SCALE
            return 0

        lax.fori_loop(0, C, row_body, 0)

    gather_start(0, 0)
    for c in range(NBUF):
        gather_wait(c, c)
        if c + 1 < NCHUNK:
            if c + 1 >= NBUF:
                out_wait(c + 1 - NBUF, (c + 1) % NBUF)
            gather_start(c + 1, (c + 1) % NBUF)
        scale_buf(c)
        out_start(c, c)

    n_groups = (NCHUNK - 1 - NBUF) // NBUF

    def group_body(j, _):
        c0 = NBUF + j * NBUF
        for b in range(NBUF):
            c = c0 + b
            gather_wait(c, b)
            scale_buf(b)
            nb = (b + 1) % NBUF
            out_wait(c - (NBUF - 1), nb)
            gather_start(c + 1, nb)
            out_start(c, b)
        return 0

    lax.fori_loop(0, n_groups, group_body, 0)

    for c in range(NBUF + n_groups * NBUF, NCHUNK):
        b = c % NBUF
        gather_wait(c, b)
        if c + 1 < NCHUNK:
            nb = (c + 1) % NBUF
            out_wait(c + 1 - NBUF, nb)
            gather_start(c + 1, nb)
            scale_buf(b)
            out_start(c, b)
        else:
            q = C // 4
            for h in range(4):
                def piece_body(r, _):
                    for j in range(D_MODEL // L):
                        sl = pl.ds(j * L, L)
                        rows_v[b, r, sl] = rows_v[b, r, sl] * SCALE
                    return 0

                lax.fori_loop(h * q, (h + 1) * q, piece_body, 0)
                pltpu.async_copy(
                    rows_v.at[b].at[pl.ds(h * q, q)],
                    out_hbm.at[pl.ds(base + c * C + h * q, q)],
                    osem,
                )

    for c in range(NCHUNK - NBUF, NCHUNK - 1):
        out_wait(c, c % NBUF)
    bl = (NCHUNK - 1) % NBUF
    q = C // 4
    for h in range(4):
        pltpu.make_async_copy(
            rows_v.at[bl].at[pl.ds(h * q, q)],
            out_hbm.at[pl.ds(base + (NCHUNK - 1) * C + h * q, q)],
            osem,
        ).wait()


def kernel(x, table):
    idx = x.astype(jnp.int32).reshape(B_TOTAL)
    out = _emb_lookup(table, idx)
    return out.reshape(x.shape + (D_MODEL,))


# speedup vs baseline: 1.2192x; 1.2192x over previous
import functools
import math

import jax
import jax.numpy as jnp
from jax import lax
from jax.experimental import pallas as pl
from jax.experimental.pallas import tpu as pltpu
from jax.experimental.pallas import tpu_sc as plsc

D_MODEL = 1024
SCALE = math.sqrt(D_MODEL)
L = 16
NC, NS = 2, 16
NW = NC * NS
B_TOTAL = 4 * 4096
BPW = B_TOTAL // NW
CH = 16
NSTEP = BPW // CH
NB = 6
PF = 2

_mesh = plsc.VectorSubcoreMesh(core_axis_name="c", subcore_axis_name="s")


@functools.partial(
    pl.kernel,
    mesh=_mesh,
    out_type=jax.ShapeDtypeStruct((B_TOTAL, D_MODEL), jnp.float32),
    scratch_types=[
        pltpu.VMEM((BPW,), jnp.int32),
        pltpu.VMEM((NB, CH, D_MODEL), jnp.float32),
        pltpu.SemaphoreType.DMA,
        pltpu.SemaphoreType.DMA,
    ],
)
def _emb_lookup(table_hbm, idx_hbm, out_hbm, idx_v, rows_v, gsem, osem):
    wid = lax.axis_index("s") * NC + lax.axis_index("c")
    base = wid * BPW

    pltpu.sync_copy(idx_hbm.at[pl.ds(base, BPW)], idx_v)

    def gather_start(s, b):
        return pltpu.async_copy(
            table_hbm.at[idx_v.at[pl.ds(s * CH, CH)]], rows_v.at[b], gsem
        )

    def gather_wait(s, b):
        pltpu.make_async_copy(
            table_hbm.at[idx_v.at[pl.ds(s * CH, CH)]], rows_v.at[b], gsem
        ).wait()

    def out_start(s, b):
        return pltpu.async_copy(
            rows_v.at[b], out_hbm.at[pl.ds(base + s * CH, CH)], osem
        )

    def out_wait(s, b):
        pltpu.make_async_copy(
            rows_v.at[b], out_hbm.at[pl.ds(base + s * CH, CH)], osem
        ).wait()

    def scale_buf(b):
        def row_body(r, _):
            for j in range(D_MODEL // L):
                sl = pl.ds(j * L, L)
                rows_v[b, r, sl] = rows_v[b, r, sl] * SCALE
            return 0

        lax.fori_loop(0, CH, row_body, 0)

    def step_body(s, b):
        gather_wait(s, b)
        t = s + PF
        tb = (b + PF) % NB
        if isinstance(s, int) and t - NB < 0:
            pass
        else:
            out_wait(t - NB, tb)
        gather_start(t, tb)
        scale_buf(b)
        out_start(s, b)

    for t in range(PF):
        gather_start(t, t)

    for s in range(NB):
        step_body(s, s)

    n_groups = (NSTEP - PF - NB) // NB

    def group_body(j, _):
        s0 = NB + j * NB
        for b in range(NB):
            step_body(s0 + b, b)
        return 0

    lax.fori_loop(0, n_groups, group_body, 0)

    tail0 = NB + n_groups * NB
    for s in range(tail0, NSTEP):
        b = s % NB
        gather_wait(s, b)
        if s == NSTEP - 1:
            q = CH // 2
            for h in range(2):
                def piece_body(r, _):
                    for j in range(D_MODEL // L):
                        sl = pl.ds(j * L, L)
                        rows_v[b, r, sl] = rows_v[b, r, sl] * SCALE
                    return 0

                lax.fori_loop(h * q, (h + 1) * q, piece_body, 0)
                pltpu.async_copy(
                    rows_v.at[b].at[pl.ds(h * q, q)],
                    out_hbm.at[pl.ds(base + s * CH + h * q, q)],
                    osem,
                )
        else:
            scale_buf(b)
            out_start(s, b)

    for s in range(NSTEP - NB, NSTEP - 1):
        out_wait(s, s % NB)
    bl = (NSTEP - 1) % NB
    q = CH // 2
    for h in range(2):
        pltpu.make_async_copy(
            rows_v.at[bl].at[pl.ds(h * q, q)],
            out_hbm.at[pl.ds(base + (NSTEP - 1) * CH + h * q, q)],
            osem,
        ).wait()


def kernel(x, table):
    idx = x.astype(jnp.int32).reshape(B_TOTAL)
    out = _emb_lookup(table, idx)
    return out.reshape(x.shape + (D_MODEL,))


# speedup vs baseline: 1.2592x; 1.0328x over previous
import functools
import math

import jax
import jax.numpy as jnp
from jax import lax
from jax.experimental import pallas as pl
from jax.experimental.pallas import tpu as pltpu
from jax.experimental.pallas import tpu_sc as plsc

D_MODEL = 1024
SCALE = math.sqrt(D_MODEL)
L = 16
NC, NS = 2, 16
NW = NC * NS
B_TOTAL = 4 * 4096
BPW = B_TOTAL // NW
C = 32
NCHUNK = BPW // C
NBUF = 3

_mesh = plsc.VectorSubcoreMesh(core_axis_name="c", subcore_axis_name="s")


@functools.partial(
    pl.kernel,
    mesh=_mesh,
    out_type=jax.ShapeDtypeStruct((B_TOTAL, D_MODEL), jnp.float32),
    scratch_types=[
        pltpu.VMEM((BPW,), jnp.int32),
        pltpu.VMEM((NBUF, C, D_MODEL), jnp.float32),
        pltpu.SemaphoreType.DMA,
        pltpu.SemaphoreType.DMA,
    ],
)
def _emb_lookup(table_hbm, idx_hbm, out_hbm, idx_v, rows_v, gsem, osem):
    wid = lax.axis_index("s") * NC + lax.axis_index("c")
    base = wid * BPW

    pltpu.sync_copy(idx_hbm.at[pl.ds(base, BPW)], idx_v)

    def gather_start(c, b):
        return pltpu.async_copy(
            table_hbm.at[idx_v.at[pl.ds(c * C, C)]], rows_v.at[b], gsem
        )

    def gather_wait(c, b):
        pltpu.make_async_copy(
            table_hbm.at[idx_v.at[pl.ds(c * C, C)]], rows_v.at[b], gsem
        ).wait()

    def out_start(c, b):
        return pltpu.async_copy(
            rows_v.at[b], out_hbm.at[pl.ds(base + c * C, C)], osem
        )

    def out_wait(c, b):
        pltpu.make_async_copy(
            rows_v.at[b], out_hbm.at[pl.ds(base + c * C, C)], osem
        ).wait()

    def scale_buf(b):
        def row_body(r, _):
            for j in range(D_MODEL // L):
                sl = pl.ds(j * L, L)
                rows_v[b, r, sl] = rows_v[b, r, sl] * SCALE
            return 0

        lax.fori_loop(0, C, row_body, 0)

    gather_start(0, 0)
    for c in range(NBUF):
        gather_wait(c, c)
        if c + 1 < NCHUNK:
            if c + 1 >= NBUF:
                out_wait(c + 1 - NBUF, (c + 1) % NBUF)
            gather_start(c + 1, (c + 1) % NBUF)
        scale_buf(c)
        out_start(c, c)

    n_groups = (NCHUNK - 1 - NBUF) // NBUF

    def group_body(j, _):
        c0 = NBUF + j * NBUF
        for b in range(NBUF):
            c = c0 + b
            gather_wait(c, b)
            nb = (b + 1) % NBUF
            out_wait(c - (NBUF - 1), nb)
            gather_start(c + 1, nb)
            scale_buf(b)
            out_start(c, b)
        return 0

    lax.fori_loop(0, n_groups, group_body, 0)

    for c in range(NBUF + n_groups * NBUF, NCHUNK):
        b = c % NBUF
        gather_wait(c, b)
        if c + 1 < NCHUNK:
            nb = (c + 1) % NBUF
            out_wait(c + 1 - NBUF, nb)
            gather_start(c + 1, nb)
            scale_buf(b)
            out_start(c, b)
        else:
            q = C // 4
            for h in range(4):
                def piece_body(r, _):
                    for j in range(D_MODEL // L):
                        sl = pl.ds(j * L, L)
                        rows_v[b, r, sl] = rows_v[b, r, sl] * SCALE
                    return 0

                lax.fori_loop(h * q, (h + 1) * q, piece_body, 0)
                pltpu.async_copy(
                    rows_v.at[b].at[pl.ds(h * q, q)],
                    out_hbm.at[pl.ds(base + c * C + h * q, q)],
                    osem,
                )

    for c in range(NCHUNK - NBUF, NCHUNK - 1):
        out_wait(c, c % NBUF)
    bl = (NCHUNK - 1) % NBUF
    q = C // 4
    for h in range(4):
        pltpu.make_async_copy(
            rows_v.at[bl].at[pl.ds(h * q, q)],
            out_hbm.at[pl.ds(base + (NCHUNK - 1) * C + h * q, q)],
            osem,
        ).wait()


def kernel(x, table):
    idx = x.astype(jnp.int32).reshape(B_TOTAL)
    out = _emb_lookup(table, idx)
    return out.reshape(x.shape + (D_MODEL,))
